# Initial kernel scaffold; baseline (speedup 1.0000x reference)
import jax
import jax.numpy as jnp
from jax.experimental import pallas as pl


def kernel(nodes, edges, senders, receivers, num_processing_steps, params):
    raise NotImplementedError("write your pallas kernel here")


# speedup vs baseline: 1.7499x; 1.7499x over previous
import functools

import jax
import jax.numpy as jnp
from jax import lax
from jax.experimental import pallas as pl


LAT = 64


def _leaky(x):
    return jnp.where(x > 0, x, 0.01 * x)


def _ln_rows(h, g, b):
    m = jnp.mean(h, axis=-1, keepdims=True)
    d = h - m
    v = jnp.mean(d * d, axis=-1, keepdims=True)
    return d * jax.lax.rsqrt(v + 1e-5) * g + b


def _edge_mlp_body(x1_ref, x2_ref, c_ref, m_ref, w1_ref, b1_ref, w2_ref,
                   b2_ref, w3_ref, b3_ref, g_ref, beta_ref, o_ref):
    h = jnp.dot(x1_ref[...], m_ref[...], preferred_element_type=jnp.float32)
    h = h + x2_ref[...]
    if c_ref is not None:
        h = h + c_ref[...]
    h = _leaky(h)
    h = _leaky(jnp.dot(h, w1_ref[...], preferred_element_type=jnp.float32) + b1_ref[...])
    h = _leaky(jnp.dot(h, w2_ref[...], preferred_element_type=jnp.float32) + b2_ref[...])
    h = jnp.dot(h, w3_ref[...], preferred_element_type=jnp.float32) + b3_ref[...]
    o_ref[...] = _ln_rows(h, g_ref[...], beta_ref[...])


def _edge_mlp(x1, x2, c, M, tail_params, ln, block_rows=4000):
    E, K = x1.shape
    grid = (E // block_rows,)
    (w1, b1), (w2, b2), (w3, b3) = tail_params
    g, beta = ln
    row_spec = pl.BlockSpec((block_rows, LAT), lambda i: (i, 0))
    x1_spec = pl.BlockSpec((block_rows, K), lambda i: (i, 0))
    full = lambda a: pl.BlockSpec(a.shape, lambda i: (0,) * a.ndim)
    small = [M, w1.reshape(LAT, LAT), b1.reshape(1, LAT), w2, b2.reshape(1, LAT),
             w3, b3.reshape(1, LAT), g.reshape(1, LAT), beta.reshape(1, LAT)]
    body = _edge_mlp_body if c is not None else (
        lambda x1r, x2r, *rest: _edge_mlp_body(x1r, x2r, None, *rest))
    args = [x1, x2] + ([c] if c is not None else []) + small
    specs = [x1_spec, row_spec] + ([row_spec] if c is not None else []) + [full(a) for a in small]
    return pl.pallas_call(
        body,
        grid=grid,
        in_specs=specs,
        out_specs=row_spec,
        out_shape=jax.ShapeDtypeStruct((E, LAT), jnp.float32),
    )(*args)


def _enc_node_body(nodes_ref, agg0_ref, vn_ref, va_ref, b0_ref, w1_ref, b1_ref,
                   w2_ref, b2_ref, w3_ref, b3_ref, g_ref, beta_ref,
                   wg0s_ref, wg0r_ref, wsn_ref, wrn_ref, uen_ref, uga_ref,
                   bu0_ref,
                   en_ref, a0_ref, b0out_ref, p2_ref, q2_ref, cn_ref):
    dot = lambda a, b: jnp.dot(a, b, preferred_element_type=jnp.float32)
    h = dot(nodes_ref[...], vn_ref[...]) + dot(agg0_ref[...], va_ref[...]) + b0_ref[...]
    h = _leaky(h)
    h = _leaky(dot(h, w1_ref[...]) + b1_ref[...])
    h = _leaky(dot(h, w2_ref[...]) + b2_ref[...])
    h = dot(h, w3_ref[...]) + b3_ref[...]
    en = _ln_rows(h, g_ref[...], beta_ref[...])
    en_ref[...] = en
    p2 = dot(en, wg0s_ref[...])
    q2 = dot(en, wg0r_ref[...])
    p2_ref[...] = p2
    q2_ref[...] = q2
    a0_ref[...] = dot(en, wsn_ref[...])
    b0out_ref[...] = dot(en, wrn_ref[...])
    cn_ref[...] = dot(en, uen_ref[...]) + dot(agg0_ref[...], uga_ref[...]) + bu0_ref[...]


def _step_node_body(ln_ref, p0_ref, p1_ref, cn_ref, uln_ref, uagg_ref,
                    w1_ref, b1_ref, w2_ref, b2_ref, w3_ref, b3_ref,
                    g_ref, beta_ref, wsn_ref, wrn_ref,
                    lnout_ref, aout_ref, bout_ref):
    dot = lambda a, b: jnp.dot(a, b, preferred_element_type=jnp.float32)
    agg = p0_ref[...] + p1_ref[...]
    h = dot(ln_ref[...], uln_ref[...]) + dot(agg, uagg_ref[...]) + cn_ref[...]
    h = _leaky(h)
    h = _leaky(dot(h, w1_ref[...]) + b1_ref[...])
    h = _leaky(dot(h, w2_ref[...]) + b2_ref[...])
    h = dot(h, w3_ref[...]) + b3_ref[...]
    ln2 = _ln_rows(h, g_ref[...], beta_ref[...])
    lnout_ref[...] = ln2
    aout_ref[...] = dot(ln2, wsn_ref[...])
    bout_ref[...] = dot(ln2, wrn_ref[...])


def _dec_body(ln_ref, w0_ref, b0_ref, w1_ref, b1_ref, w2_ref, b2_ref,
              w3_ref, b3_ref, o_ref):
    dot = lambda a, b: jnp.dot(a, b, preferred_element_type=jnp.float32)
    h = _leaky(dot(ln_ref[...], w0_ref[...]) + b0_ref[...])
    h = _leaky(dot(h, w1_ref[...]) + b1_ref[...])
    h = _leaky(dot(h, w2_ref[...]) + b2_ref[...])
    o_ref[...] = dot(h, w3_ref[...]) + b3_ref[...]


def _enc_tables_body(nodes_ref, ts_ref, tr_ref, pe_ref, qe_ref):
    dot = lambda a, b: jnp.dot(a, b, preferred_element_type=jnp.float32)
    pe_ref[...] = dot(nodes_ref[...], ts_ref[...])
    qe_ref[...] = dot(nodes_ref[...], tr_ref[...])


def _full_call(body, args, out_shapes):
    full = lambda a: pl.BlockSpec(a.shape, lambda: (0,) * a.ndim)
    return pl.pallas_call(
        body,
        in_specs=[full(a) for a in args],
        out_specs=[pl.BlockSpec(s.shape, lambda: (0,) * len(s.shape)) for s in out_shapes],
        out_shape=out_shapes,
    )(*args)


def _gather_combine(tab_a, tab_b, senders, receivers):
    return jnp.take(tab_a, senders, axis=0) + jnp.take(tab_b, receivers, axis=0)


def _segment_sum(vals, receivers, n):
    return jax.ops.segment_sum(vals, receivers, num_segments=n)


def kernel(nodes, edges, senders, receivers, num_processing_steps, params):
    N = nodes.shape[0]
    p = params

    (We0, be0) = p['edge_enc_mlp'][0]
    T_e, T_s, T_r = We0[:16], We0[16:144], We0[144:272]
    Pe, Qe = _full_call(
        _enc_tables_body, [nodes, T_s, T_r],
        [jax.ShapeDtypeStruct((N, LAT), jnp.float32)] * 2)
    g0 = _gather_combine(Pe, Qe, senders, receivers)
    g0 = g0 + be0.reshape(1, LAT)
    ee = _edge_mlp(edges, g0, None, T_e, p['edge_enc_mlp'][1:], p['edge_enc_ln'])
    agg0 = _segment_sum(ee, receivers, N)

    (Wn0, bn0) = p['node_enc_mlp'][0]
    (Wp0, bp0) = p['edge_proc_mlp'][0]
    W_sn, W_rn, W_le = Wp0[0:64], Wp0[64:128], Wp0[128:192]
    W_g0s, W_g0r, W_ee = Wp0[192:256], Wp0[256:320], Wp0[320:384]
    (Un0, bu0) = p['node_proc_mlp'][0]
    U_ln, U_agg, U_en, U_ga = Un0[0:64], Un0[64:128], Un0[128:192], Un0[192:256]
    (w1n, b1n), (w2n, b2n), (w3n, b3n) = p['node_enc_mlp'][1:]
    gn, betan = p['node_enc_ln']
    en, a0, b0, P2, Q2, c_n = _full_call(
        _enc_node_body,
        [nodes, agg0, Wn0[:128], Wn0[128:], bn0.reshape(1, LAT),
         w1n, b1n.reshape(1, LAT), w2n, b2n.reshape(1, LAT), w3n,
         b3n.reshape(1, LAT), gn.reshape(1, LAT), betan.reshape(1, LAT),
         W_g0s, W_g0r, W_sn, W_rn, U_en, U_ga, bu0.reshape(1, LAT)],
        [jax.ShapeDtypeStruct((N, LAT), jnp.float32)] * 6)

    g2 = _gather_combine(P2, Q2, senders, receivers)
    c_e = _ce_pass(ee, g2, W_ee, bp0)

    def step(_, carry):
        ln, le, a, b = carry
        g = _gather_combine(a, b, senders, receivers)
        le2 = _edge_mlp(le, g, c_e, W_le, p['edge_proc_mlp'][1:], p['edge_proc_ln'])
        agg = _segment_sum(le2, receivers, N)
        zero = jnp.zeros_like(agg)
        (w1, b1), (w2, b2), (w3, b3) = p['node_proc_mlp'][1:]
        gp, betap = p['node_proc_ln']
        ln2, a2, b2_ = _full_call(
            _step_node_body,
            [ln, agg, zero, c_n, U_ln, U_agg,
             w1, b1.reshape(1, LAT), w2, b2.reshape(1, LAT), w3,
             b3.reshape(1, LAT), gp.reshape(1, LAT), betap.reshape(1, LAT),
             W_sn, W_rn],
            [jax.ShapeDtypeStruct((N, LAT), jnp.float32)] * 3)
        return (ln2, le2, a2, b2_)

    ln, le, _, _ = lax.fori_loop(0, num_processing_steps, step, (en, ee, a0, b0))

    (Wd0, bd0), (wd1, bd1), (wd2, bd2), (wd3, bd3) = p['dec_mlp']
    D_OUT = wd3.shape[1]
    dec = pl.pallas_call(
        _dec_body,
        grid=(N // 2000,),
        in_specs=[pl.BlockSpec((2000, LAT), lambda i: (i, 0))] +
                 [pl.BlockSpec(a.shape, lambda i: (0,) * a.ndim) for a in
                  [Wd0, bd0.reshape(1, LAT), wd1, bd1.reshape(1, LAT),
                   wd2, bd2.reshape(1, LAT), wd3, bd3.reshape(1, D_OUT)]],
        out_specs=pl.BlockSpec((2000, D_OUT), lambda i: (i, 0)),
        out_shape=jax.ShapeDtypeStruct((N, D_OUT), jnp.float32),
    )(ln, Wd0, bd0.reshape(1, LAT), wd1, bd1.reshape(1, LAT), wd2,
      bd2.reshape(1, LAT), wd3, bd3.reshape(1, D_OUT))
    return dec


def _ce_body(ee_ref, g2_ref, wee_ref, bp0_ref, o_ref):
    o_ref[...] = (jnp.dot(ee_ref[...], wee_ref[...],
                          preferred_element_type=jnp.float32)
                  + g2_ref[...] + bp0_ref[...])


def _ce_pass(ee, g2, W_ee, bp0, block_rows=4000):
    E = ee.shape[0]
    row_spec = pl.BlockSpec((block_rows, LAT), lambda i: (i, 0))
    return pl.pallas_call(
        _ce_body,
        grid=(E // block_rows,),
        in_specs=[row_spec, row_spec,
                  pl.BlockSpec((LAT, LAT), lambda i: (0, 0)),
                  pl.BlockSpec((1, LAT), lambda i: (0, 0))],
        out_specs=row_spec,
        out_shape=jax.ShapeDtypeStruct((E, LAT), jnp.float32),
    )(ee, g2, W_ee, bp0.reshape(1, LAT))


# speedup vs baseline: 4.6396x; 2.6514x over previous
import functools

import jax
import jax.numpy as jnp
from jax import lax
from jax.experimental import pallas as pl
from jax.experimental.pallas import tpu as pltpu
from jax.experimental.pallas import tpu_sc as plsc


LAT = 64
NC = 2
NS = 16
NW = NC * NS


def _leaky(x):
    return jnp.where(x > 0, x, 0.01 * x)


def _ln_rows(h, g, b):
    m = jnp.mean(h, axis=-1, keepdims=True)
    d = h - m
    v = jnp.mean(d * d, axis=-1, keepdims=True)
    return d * jax.lax.rsqrt(v + 1e-5) * g + b


_CHUNK = 128
_GSUB = 4
_GRP = _CHUNK * _GSUB


def _sc_gather_body(tabA, tabB, senders, receivers, ga, gb,
                    sidx, ridx, bufA, bufB, semA, semB, semi):
    c = lax.axis_index("c")
    s = lax.axis_index("s")
    wid = c * NS + s
    E = senders.shape[0]
    ngrp = E // _GRP
    niter = (ngrp + NW - 1) // NW

    def group(j, _):
        g = wid + j * NW

        @pl.when(g < ngrp)
        def _():
            base = g * _GRP
            cpi1 = pltpu.async_copy(senders.at[pl.ds(base, _GRP)], sidx, semi)
            cpi2 = pltpu.async_copy(receivers.at[pl.ds(base, _GRP)], ridx, semi)
            cpi1.wait()
            cpi2.wait()
            cps = []
            for k in range(_GSUB):
                cps.append(pltpu.async_copy(
                    tabA.at[sidx.at[pl.ds(k * _CHUNK, _CHUNK)]],
                    bufA.at[pl.ds(k * _CHUNK, _CHUNK)], semA))
                cps.append(pltpu.async_copy(
                    tabB.at[ridx.at[pl.ds(k * _CHUNK, _CHUNK)]],
                    bufB.at[pl.ds(k * _CHUNK, _CHUNK)], semB))
            for cp in cps:
                cp.wait()
            cpo1 = pltpu.async_copy(bufA, ga.at[pl.ds(base, _GRP)], semA)
            cpo2 = pltpu.async_copy(bufB, gb.at[pl.ds(base, _GRP)], semB)
            cpo1.wait()
            cpo2.wait()
        return 0

    lax.fori_loop(0, niter, group, 0)


def _sc_gather(tabA, tabB, senders, receivers):
    E = senders.shape[0]
    mesh = plsc.VectorSubcoreMesh(core_axis_name="c", subcore_axis_name="s")
    out = jax.ShapeDtypeStruct((E, LAT), jnp.float32)
    return pl.kernel(
        _sc_gather_body,
        out_type=(out, out),
        mesh=mesh,
        scratch_types=[
            pltpu.VMEM((_GRP,), jnp.int32),
            pltpu.VMEM((_GRP,), jnp.int32),
            pltpu.VMEM((_GRP, LAT), jnp.float32),
            pltpu.VMEM((_GRP, LAT), jnp.float32),
            pltpu.SemaphoreType.DMA,
            pltpu.SemaphoreType.DMA,
            pltpu.SemaphoreType.DMA,
        ],
        compiler_params=pltpu.CompilerParams(use_tc_tiling_on_sc=False),
    )(tabA, tabB, senders, receivers)


def _sc_scatter_body(vals, receivers, zeros, out, ridx, vbuf, acc, sem):
    c = lax.axis_index("c")
    s = lax.axis_index("s")
    wid = c * NS + s
    E = receivers.shape[0]
    N = zeros.shape[0]
    rows = N // NS
    nchunk = E // _CHUNK
    niter = (nchunk + NW - 1) // NW

    pltpu.sync_copy(zeros.at[pl.ds(s * rows, rows)], acc.at[pl.ds(s * rows, rows)])
    plsc.subcore_barrier()

    def chunk(j, _):
        ch = wid + j * NW

        @pl.when(ch < nchunk)
        def _():
            base = ch * _CHUNK
            cpi = pltpu.async_copy(receivers.at[pl.ds(base, _CHUNK)], ridx, sem)
            cpv = pltpu.async_copy(vals.at[pl.ds(base, _CHUNK)], vbuf, sem)
            cpi.wait()
            cpv.wait()
            pltpu.sync_copy(vbuf, acc.at[ridx], add=True)
        return 0

    lax.fori_loop(0, niter, chunk, 0)
    plsc.subcore_barrier()
    pltpu.sync_copy(acc.at[pl.ds(s * rows, rows)],
                    out.at[c].at[pl.ds(s * rows, rows)])


def _sc_scatter(vals, receivers, zeros):
    N = zeros.shape[0]
    mesh = plsc.VectorSubcoreMesh(core_axis_name="c", subcore_axis_name="s")
    return pl.kernel(
        _sc_scatter_body,
        out_type=jax.ShapeDtypeStruct((NC, N, LAT), jnp.float32),
        mesh=mesh,
        scratch_types=[
            pltpu.VMEM((_CHUNK,), jnp.int32),
            pltpu.VMEM((_CHUNK, LAT), jnp.float32),
            pltpu.VMEM_SHARED((N, LAT), jnp.float32),
            pltpu.SemaphoreType.DMA,
        ],
        compiler_params=pltpu.CompilerParams(use_tc_tiling_on_sc=False),
    )(vals, receivers, zeros)


def _edge_mlp_body(x1_ref, ga_ref, gb_ref, c_ref, m_ref, w1_ref, b1_ref,
                   w2_ref, b2_ref, w3_ref, b3_ref, g_ref, beta_ref, o_ref):
    h = jnp.dot(x1_ref[...], m_ref[...], preferred_element_type=jnp.float32)
    h = h + ga_ref[...] + gb_ref[...] + c_ref[...]
    h = _leaky(h)
    h = _leaky(jnp.dot(h, w1_ref[...], preferred_element_type=jnp.float32) + b1_ref[...])
    h = _leaky(jnp.dot(h, w2_ref[...], preferred_element_type=jnp.float32) + b2_ref[...])
    h = jnp.dot(h, w3_ref[...], preferred_element_type=jnp.float32) + b3_ref[...]
    o_ref[...] = _ln_rows(h, g_ref[...], beta_ref[...])


def _edge_mlp(x1, ga, gb, c, M, tail_params, ln, block_rows=4000):
    E, K = x1.shape
    grid = (E // block_rows,)
    (w1, b1), (w2, b2), (w3, b3) = tail_params
    g, beta = ln
    row_spec = pl.BlockSpec((block_rows, LAT), lambda i: (i, 0))
    x1_spec = pl.BlockSpec((block_rows, K), lambda i: (i, 0))
    c_spec = (row_spec if c.shape[0] == E
              else pl.BlockSpec((1, LAT), lambda i: (0, 0)))
    full = lambda a: pl.BlockSpec(a.shape, lambda i: (0,) * a.ndim)
    small = [M, w1, b1.reshape(1, LAT), w2, b2.reshape(1, LAT),
             w3, b3.reshape(1, LAT), g.reshape(1, LAT), beta.reshape(1, LAT)]
    return pl.pallas_call(
        _edge_mlp_body,
        grid=grid,
        in_specs=[x1_spec, row_spec, row_spec, c_spec] + [full(a) for a in small],
        out_specs=row_spec,
        out_shape=jax.ShapeDtypeStruct((E, LAT), jnp.float32),
    )(x1, ga, gb, c, *small)


def _enc_node_body(nodes_ref, p0_ref, p1_ref, vn_ref, va_ref, b0_ref, w1_ref,
                   b1_ref, w2_ref, b2_ref, w3_ref, b3_ref, g_ref, beta_ref,
                   wg0s_ref, wg0r_ref, wsn_ref, wrn_ref, uen_ref, uga_ref,
                   bu0_ref,
                   en_ref, a0_ref, b0out_ref, p2_ref, q2_ref, cn_ref):
    dot = lambda a, b: jnp.dot(a, b, preferred_element_type=jnp.float32)
    agg0 = p0_ref[...] + p1_ref[...]
    h = dot(nodes_ref[...], vn_ref[...]) + dot(agg0, va_ref[...]) + b0_ref[...]
    h = _leaky(h)
    h = _leaky(dot(h, w1_ref[...]) + b1_ref[...])
    h = _leaky(dot(h, w2_ref[...]) + b2_ref[...])
    h = dot(h, w3_ref[...]) + b3_ref[...]
    en = _ln_rows(h, g_ref[...], beta_ref[...])
    en_ref[...] = en
    p2_ref[...] = dot(en, wg0s_ref[...])
    q2_ref[...] = dot(en, wg0r_ref[...])
    a0_ref[...] = dot(en, wsn_ref[...])
    b0out_ref[...] = dot(en, wrn_ref[...])
    cn_ref[...] = dot(en, uen_ref[...]) + dot(agg0, uga_ref[...]) + bu0_ref[...]


def _step_node_body(ln_ref, p0_ref, p1_ref, cn_ref, uln_ref, uagg_ref,
                    w1_ref, b1_ref, w2_ref, b2_ref, w3_ref, b3_ref,
                    g_ref, beta_ref, wsn_ref, wrn_ref,
                    lnout_ref, aout_ref, bout_ref):
    dot = lambda a, b: jnp.dot(a, b, preferred_element_type=jnp.float32)
    agg = p0_ref[...] + p1_ref[...]
    h = dot(ln_ref[...], uln_ref[...]) + dot(agg, uagg_ref[...]) + cn_ref[...]
    h = _leaky(h)
    h = _leaky(dot(h, w1_ref[...]) + b1_ref[...])
    h = _leaky(dot(h, w2_ref[...]) + b2_ref[...])
    h = dot(h, w3_ref[...]) + b3_ref[...]
    ln2 = _ln_rows(h, g_ref[...], beta_ref[...])
    lnout_ref[...] = ln2
    aout_ref[...] = dot(ln2, wsn_ref[...])
    bout_ref[...] = dot(ln2, wrn_ref[...])


def _dec_body(ln_ref, w0_ref, b0_ref, w1_ref, b1_ref, w2_ref, b2_ref,
              w3_ref, b3_ref, o_ref):
    dot = lambda a, b: jnp.dot(a, b, preferred_element_type=jnp.float32)
    h = _leaky(dot(ln_ref[...], w0_ref[...]) + b0_ref[...])
    h = _leaky(dot(h, w1_ref[...]) + b1_ref[...])
    h = _leaky(dot(h, w2_ref[...]) + b2_ref[...])
    o_ref[...] = dot(h, w3_ref[...]) + b3_ref[...]


def _enc_tables_body(nodes_ref, ts_ref, tr_ref, pe_ref, qe_ref):
    dot = lambda a, b: jnp.dot(a, b, preferred_element_type=jnp.float32)
    pe_ref[...] = dot(nodes_ref[...], ts_ref[...])
    qe_ref[...] = dot(nodes_ref[...], tr_ref[...])


def _full_call(body, args, out_shapes):
    full = lambda a: pl.BlockSpec(a.shape, lambda: (0,) * a.ndim)
    return pl.pallas_call(
        body,
        in_specs=[full(a) for a in args],
        out_specs=[pl.BlockSpec(s.shape, lambda: (0,) * len(s.shape)) for s in out_shapes],
        out_shape=out_shapes,
    )(*args)


def _ce_body(ee_ref, g2a_ref, g2b_ref, wee_ref, bp0_ref, o_ref):
    o_ref[...] = (jnp.dot(ee_ref[...], wee_ref[...],
                          preferred_element_type=jnp.float32)
                  + g2a_ref[...] + g2b_ref[...] + bp0_ref[...])


def _ce_pass(ee, g2a, g2b, W_ee, bp0, block_rows=4000):
    E = ee.shape[0]
    row_spec = pl.BlockSpec((block_rows, LAT), lambda i: (i, 0))
    return pl.pallas_call(
        _ce_body,
        grid=(E // block_rows,),
        in_specs=[row_spec, row_spec, row_spec,
                  pl.BlockSpec((LAT, LAT), lambda i: (0, 0)),
                  pl.BlockSpec((1, LAT), lambda i: (0, 0))],
        out_specs=row_spec,
        out_shape=jax.ShapeDtypeStruct((E, LAT), jnp.float32),
    )(ee, g2a, g2b, W_ee, bp0.reshape(1, LAT))


def kernel(nodes, edges, senders, receivers, num_processing_steps, params):
    N = nodes.shape[0]
    p = params
    zeros = jnp.zeros((N, LAT), jnp.float32)

    (We0, be0) = p['edge_enc_mlp'][0]
    T_e, T_s, T_r = We0[:16], We0[16:144], We0[144:272]
    Pe, Qe = _full_call(
        _enc_tables_body, [nodes, T_s, T_r],
        [jax.ShapeDtypeStruct((N, LAT), jnp.float32)] * 2)
    ga0, gb0 = _sc_gather(Pe, Qe, senders, receivers)
    ee = _edge_mlp(edges, ga0, gb0, be0.reshape(1, LAT), T_e,
                   p['edge_enc_mlp'][1:], p['edge_enc_ln'])
    parts0 = _sc_scatter(ee, receivers, zeros)

    (Wn0, bn0) = p['node_enc_mlp'][0]
    (Wp0, bp0) = p['edge_proc_mlp'][0]
    W_sn, W_rn, W_le = Wp0[0:64], Wp0[64:128], Wp0[128:192]
    W_g0s, W_g0r, W_ee = Wp0[192:256], Wp0[256:320], Wp0[320:384]
    (Un0, bu0) = p['node_proc_mlp'][0]
    U_ln, U_agg, U_en, U_ga = Un0[0:64], Un0[64:128], Un0[128:192], Un0[192:256]
    (w1n, b1n), (w2n, b2n), (w3n, b3n) = p['node_enc_mlp'][1:]
    gn, betan = p['node_enc_ln']
    en, a0, b0, P2, Q2, c_n = _full_call(
        _enc_node_body,
        [nodes, parts0[0], parts0[1], Wn0[:128], Wn0[128:], bn0.reshape(1, LAT),
         w1n, b1n.reshape(1, LAT), w2n, b2n.reshape(1, LAT), w3n,
         b3n.reshape(1, LAT), gn.reshape(1, LAT), betan.reshape(1, LAT),
         W_g0s, W_g0r, W_sn, W_rn, U_en, U_ga, bu0.reshape(1, LAT)],
        [jax.ShapeDtypeStruct((N, LAT), jnp.float32)] * 6)

    g2a, g2b = _sc_gather(P2, Q2, senders, receivers)
    c_e = _ce_pass(ee, g2a, g2b, W_ee, bp0)

    (w1, b1), (w2, b2), (w3, b3) = p['node_proc_mlp'][1:]
    gp, betap = p['node_proc_ln']

    def step(_, carry):
        ln, le, a, b = carry
        ga, gb = _sc_gather(a, b, senders, receivers)
        le2 = _edge_mlp(le, ga, gb, c_e, W_le, p['edge_proc_mlp'][1:],
                        p['edge_proc_ln'])
        parts = _sc_scatter(le2, receivers, zeros)
        ln2, a2, b2_ = _full_call(
            _step_node_body,
            [ln, parts[0], parts[1], c_n, U_ln, U_agg,
             w1, b1.reshape(1, LAT), w2, b2.reshape(1, LAT), w3,
             b3.reshape(1, LAT), gp.reshape(1, LAT), betap.reshape(1, LAT),
             W_sn, W_rn],
            [jax.ShapeDtypeStruct((N, LAT), jnp.float32)] * 3)
        return (ln2, le2, a2, b2_)

    ln, le, _, _ = lax.fori_loop(0, num_processing_steps, step, (en, ee, a0, b0))

    (Wd0, bd0), (wd1, bd1), (wd2, bd2), (wd3, bd3) = p['dec_mlp']
    D_OUT = wd3.shape[1]
    dec = pl.pallas_call(
        _dec_body,
        grid=(N // 2000,),
        in_specs=[pl.BlockSpec((2000, LAT), lambda i: (i, 0))] +
                 [pl.BlockSpec(a.shape, lambda i: (0,) * a.ndim) for a in
                  [Wd0, bd0.reshape(1, LAT), wd1, bd1.reshape(1, LAT),
                   wd2, bd2.reshape(1, LAT), wd3, bd3.reshape(1, D_OUT)]],
        out_specs=pl.BlockSpec((2000, D_OUT), lambda i: (i, 0)),
        out_shape=jax.ShapeDtypeStruct((N, D_OUT), jnp.float32),
    )(ln, Wd0, bd0.reshape(1, LAT), wd1, bd1.reshape(1, LAT), wd2,
      bd2.reshape(1, LAT), wd3, bd3.reshape(1, D_OUT))
    return dec


# speedup vs baseline: 6.4098x; 1.3815x over previous
import functools

import jax
import jax.numpy as jnp
from jax import lax
from jax.experimental import pallas as pl
from jax.experimental.pallas import tpu as pltpu
from jax.experimental.pallas import tpu_sc as plsc


LAT = 64
NC = 2
NS = 16
NW = NC * NS


def _leaky(x):
    return jnp.where(x > 0, x, 0.01 * x)


def _ln_rows(h, g, b):
    m = jnp.mean(h, axis=-1, keepdims=True)
    d = h - m
    v = jnp.mean(d * d, axis=-1, keepdims=True)
    return d * jax.lax.rsqrt(v + 1e-5) * g + b


_CHUNK = 128
_GSUB = 4
_GRP = _CHUNK * _GSUB


def _sc_gather_body(tabA, tabB, senders, receivers, ga, gb,
                    sidx, ridx, bufA, bufB, semA, semB, semi):
    c = lax.axis_index("c")
    s = lax.axis_index("s")
    wid = c * NS + s
    E = senders.shape[0]
    ngrp = E // _GRP
    niter = (ngrp + NW - 1) // NW

    def group(j, _):
        g = wid + j * NW

        @pl.when(g < ngrp)
        def _():
            base = g * _GRP
            cpi1 = pltpu.async_copy(senders.at[pl.ds(base, _GRP)], sidx, semi)
            cpi2 = pltpu.async_copy(receivers.at[pl.ds(base, _GRP)], ridx, semi)
            cpi1.wait()
            cpi2.wait()
            cps = []
            for k in range(_GSUB):
                cps.append(pltpu.async_copy(
                    tabA.at[sidx.at[pl.ds(k * _CHUNK, _CHUNK)]],
                    bufA.at[pl.ds(k * _CHUNK, _CHUNK)], semA))
                cps.append(pltpu.async_copy(
                    tabB.at[ridx.at[pl.ds(k * _CHUNK, _CHUNK)]],
                    bufB.at[pl.ds(k * _CHUNK, _CHUNK)], semB))
            for cp in cps:
                cp.wait()
            cpo1 = pltpu.async_copy(bufA, ga.at[pl.ds(base, _GRP)], semA)
            cpo2 = pltpu.async_copy(bufB, gb.at[pl.ds(base, _GRP)], semB)
            cpo1.wait()
            cpo2.wait()
        return 0

    lax.fori_loop(0, niter, group, 0)


def _sc_gather(tabA, tabB, senders, receivers):
    E = senders.shape[0]
    mesh = plsc.VectorSubcoreMesh(core_axis_name="c", subcore_axis_name="s")
    out = jax.ShapeDtypeStruct((E, LAT), jnp.float32)
    return pl.kernel(
        _sc_gather_body,
        out_type=(out, out),
        mesh=mesh,
        scratch_types=[
            pltpu.VMEM((_GRP,), jnp.int32),
            pltpu.VMEM((_GRP,), jnp.int32),
            pltpu.VMEM((_GRP, LAT), jnp.float32),
            pltpu.VMEM((_GRP, LAT), jnp.float32),
            pltpu.SemaphoreType.DMA,
            pltpu.SemaphoreType.DMA,
            pltpu.SemaphoreType.DMA,
        ],
        compiler_params=pltpu.CompilerParams(use_tc_tiling_on_sc=False),
    )(tabA, tabB, senders, receivers)


def _sc_scatter_body(vals, receivers, zeros, out, ridx, vbuf, acc, sem):
    c = lax.axis_index("c")
    s = lax.axis_index("s")
    wid = c * NS + s
    E = receivers.shape[0]
    N = zeros.shape[0]
    rows = N // NS
    nchunk = E // _CHUNK
    niter = (nchunk + NW - 1) // NW

    pltpu.sync_copy(zeros.at[pl.ds(s * rows, rows)], acc.at[pl.ds(s * rows, rows)])
    plsc.subcore_barrier()

    def chunk(j, _):
        ch = wid + j * NW

        @pl.when(ch < nchunk)
        def _():
            base = ch * _CHUNK
            cpi = pltpu.async_copy(receivers.at[pl.ds(base, _CHUNK)], ridx, sem)
            cpv = pltpu.async_copy(vals.at[pl.ds(base, _CHUNK)], vbuf, sem)
            cpi.wait()
            cpv.wait()
            pltpu.sync_copy(vbuf, acc.at[ridx], add=True)
        return 0

    lax.fori_loop(0, niter, chunk, 0)
    plsc.subcore_barrier()
    pltpu.sync_copy(acc.at[pl.ds(s * rows, rows)],
                    out.at[c].at[pl.ds(s * rows, rows)])


def _sc_scatter(vals, receivers, zeros):
    N = zeros.shape[0]
    mesh = plsc.VectorSubcoreMesh(core_axis_name="c", subcore_axis_name="s")
    return pl.kernel(
        _sc_scatter_body,
        out_type=jax.ShapeDtypeStruct((NC, N, LAT), jnp.float32),
        mesh=mesh,
        scratch_types=[
            pltpu.VMEM((_CHUNK,), jnp.int32),
            pltpu.VMEM((_CHUNK, LAT), jnp.float32),
            pltpu.VMEM_SHARED((N, LAT), jnp.float32),
            pltpu.SemaphoreType.DMA,
        ],
        compiler_params=pltpu.CompilerParams(use_tc_tiling_on_sc=False),
    )(vals, receivers, zeros)


PK = 4
LATP = LAT * PK


def _edge_mlp_body(x1_ref, ga_ref, gb_ref, c_ref, m_ref, w1_ref, b1_ref,
                   w2_ref, b2_ref, w3_ref, b3_ref, bgrp_ref, g_ref, beta_ref,
                   o_ref):
    h = jnp.dot(x1_ref[...], m_ref[...], preferred_element_type=jnp.float32)
    h = h + ga_ref[...] + gb_ref[...] + c_ref[...]
    h = _leaky(h)
    h = _leaky(jnp.dot(h, w1_ref[...], preferred_element_type=jnp.float32) + b1_ref[...])
    h = _leaky(jnp.dot(h, w2_ref[...], preferred_element_type=jnp.float32) + b2_ref[...])
    h = jnp.dot(h, w3_ref[...], preferred_element_type=jnp.float32) + b3_ref[...]
    mu = jnp.dot(h, bgrp_ref[...], preferred_element_type=jnp.float32)
    d = h - mu
    v = jnp.dot(d * d, bgrp_ref[...], preferred_element_type=jnp.float32)
    o_ref[...] = d * jax.lax.rsqrt(v + 1e-5) * g_ref[...] + beta_ref[...]


def _edge_mlp(x1, ga, gb, c, M, tail_params, ln, block_rows=1000):
    EP, KP = x1.shape
    grid = (EP // block_rows,)
    (w1, b1), (w2, b2), (w3, b3) = tail_params
    g, beta = ln
    blk = lambda W: jnp.kron(jnp.eye(PK, dtype=W.dtype), W)
    row_spec = pl.BlockSpec((block_rows, LATP), lambda i: (i, 0))
    x1_spec = pl.BlockSpec((block_rows, KP), lambda i: (i, 0))
    c_spec = (row_spec if c.shape[0] == EP
              else pl.BlockSpec((1, LATP), lambda i: (0, 0)))
    full = lambda a: pl.BlockSpec(a.shape, lambda i: (0,) * a.ndim)
    bgrp = jnp.kron(jnp.eye(PK, dtype=jnp.float32),
                    jnp.full((LAT, LAT), 1.0 / LAT, jnp.float32))
    small = [blk(M), blk(w1), jnp.tile(b1, PK).reshape(1, LATP),
             blk(w2), jnp.tile(b2, PK).reshape(1, LATP),
             blk(w3), jnp.tile(b3, PK).reshape(1, LATP), bgrp,
             jnp.tile(g, PK).reshape(1, LATP),
             jnp.tile(beta, PK).reshape(1, LATP)]
    return pl.pallas_call(
        _edge_mlp_body,
        grid=grid,
        in_specs=[x1_spec, row_spec, row_spec, c_spec] + [full(a) for a in small],
        out_specs=row_spec,
        out_shape=jax.ShapeDtypeStruct((EP, LATP), jnp.float32),
    )(x1, ga, gb, c, *small)


def _enc_node_body(nodes_ref, p0_ref, p1_ref, vn_ref, va_ref, b0_ref, w1_ref,
                   b1_ref, w2_ref, b2_ref, w3_ref, b3_ref, g_ref, beta_ref,
                   wg0s_ref, wg0r_ref, wsn_ref, wrn_ref, uen_ref, uga_ref,
                   bu0_ref,
                   en_ref, a0_ref, b0out_ref, p2_ref, q2_ref, cn_ref):
    dot = lambda a, b: jnp.dot(a, b, preferred_element_type=jnp.float32)
    agg0 = p0_ref[...] + p1_ref[...]
    h = dot(nodes_ref[...], vn_ref[...]) + dot(agg0, va_ref[...]) + b0_ref[...]
    h = _leaky(h)
    h = _leaky(dot(h, w1_ref[...]) + b1_ref[...])
    h = _leaky(dot(h, w2_ref[...]) + b2_ref[...])
    h = dot(h, w3_ref[...]) + b3_ref[...]
    en = _ln_rows(h, g_ref[...], beta_ref[...])
    en_ref[...] = en
    p2_ref[...] = dot(en, wg0s_ref[...])
    q2_ref[...] = dot(en, wg0r_ref[...])
    a0_ref[...] = dot(en, wsn_ref[...])
    b0out_ref[...] = dot(en, wrn_ref[...])
    cn_ref[...] = dot(en, uen_ref[...]) + dot(agg0, uga_ref[...]) + bu0_ref[...]


def _step_node_body(ln_ref, p0_ref, p1_ref, cn_ref, uln_ref, uagg_ref,
                    w1_ref, b1_ref, w2_ref, b2_ref, w3_ref, b3_ref,
                    g_ref, beta_ref, wsn_ref, wrn_ref,
                    lnout_ref, aout_ref, bout_ref):
    dot = lambda a, b: jnp.dot(a, b, preferred_element_type=jnp.float32)
    agg = p0_ref[...] + p1_ref[...]
    h = dot(ln_ref[...], uln_ref[...]) + dot(agg, uagg_ref[...]) + cn_ref[...]
    h = _leaky(h)
    h = _leaky(dot(h, w1_ref[...]) + b1_ref[...])
    h = _leaky(dot(h, w2_ref[...]) + b2_ref[...])
    h = dot(h, w3_ref[...]) + b3_ref[...]
    ln2 = _ln_rows(h, g_ref[...], beta_ref[...])
    lnout_ref[...] = ln2
    aout_ref[...] = dot(ln2, wsn_ref[...])
    bout_ref[...] = dot(ln2, wrn_ref[...])


def _dec_body(ln_ref, w0_ref, b0_ref, w1_ref, b1_ref, w2_ref, b2_ref,
              w3_ref, b3_ref, o_ref):
    dot = lambda a, b: jnp.dot(a, b, preferred_element_type=jnp.float32)
    h = _leaky(dot(ln_ref[...], w0_ref[...]) + b0_ref[...])
    h = _leaky(dot(h, w1_ref[...]) + b1_ref[...])
    h = _leaky(dot(h, w2_ref[...]) + b2_ref[...])
    o_ref[...] = dot(h, w3_ref[...]) + b3_ref[...]


def _enc_tables_body(nodes_ref, ts_ref, tr_ref, pe_ref, qe_ref):
    dot = lambda a, b: jnp.dot(a, b, preferred_element_type=jnp.float32)
    pe_ref[...] = dot(nodes_ref[...], ts_ref[...])
    qe_ref[...] = dot(nodes_ref[...], tr_ref[...])


def _full_call(body, args, out_shapes):
    full = lambda a: pl.BlockSpec(a.shape, lambda: (0,) * a.ndim)
    return pl.pallas_call(
        body,
        in_specs=[full(a) for a in args],
        out_specs=[pl.BlockSpec(s.shape, lambda: (0,) * len(s.shape)) for s in out_shapes],
        out_shape=out_shapes,
    )(*args)


def _ce_body(ee_ref, g2a_ref, g2b_ref, wee_ref, bp0_ref, o_ref):
    o_ref[...] = (jnp.dot(ee_ref[...], wee_ref[...],
                          preferred_element_type=jnp.float32)
                  + g2a_ref[...] + g2b_ref[...] + bp0_ref[...])


def _ce_pass(ee, g2a, g2b, W_ee, bp0, block_rows=1000):
    EP = ee.shape[0]
    row_spec = pl.BlockSpec((block_rows, LATP), lambda i: (i, 0))
    return pl.pallas_call(
        _ce_body,
        grid=(EP // block_rows,),
        in_specs=[row_spec, row_spec, row_spec,
                  pl.BlockSpec((LATP, LATP), lambda i: (0, 0)),
                  pl.BlockSpec((1, LATP), lambda i: (0, 0))],
        out_specs=row_spec,
        out_shape=jax.ShapeDtypeStruct((EP, LATP), jnp.float32),
    )(ee, g2a, g2b, jnp.kron(jnp.eye(PK, dtype=W_ee.dtype), W_ee),
      jnp.tile(bp0, PK).reshape(1, LATP))


def kernel(nodes, edges, senders, receivers, num_processing_steps, params):
    N = nodes.shape[0]
    E = senders.shape[0]
    EP = E // PK
    p = params
    zeros = jnp.zeros((N, LAT), jnp.float32)
    pack = lambda x: x.reshape(EP, x.shape[1] * PK)
    unpack = lambda x: x.reshape(E, LAT)

    (We0, be0) = p['edge_enc_mlp'][0]
    T_e, T_s, T_r = We0[:16], We0[16:144], We0[144:272]
    Pe, Qe = _full_call(
        _enc_tables_body, [nodes, T_s, T_r],
        [jax.ShapeDtypeStruct((N, LAT), jnp.float32)] * 2)
    ga0, gb0 = _sc_gather(Pe, Qe, senders, receivers)
    ee = _edge_mlp(pack(edges), pack(ga0), pack(gb0),
                   jnp.tile(be0, PK).reshape(1, LATP), T_e,
                   p['edge_enc_mlp'][1:], p['edge_enc_ln'])
    parts0 = _sc_scatter(unpack(ee), receivers, zeros)

    (Wn0, bn0) = p['node_enc_mlp'][0]
    (Wp0, bp0) = p['edge_proc_mlp'][0]
    W_sn, W_rn, W_le = Wp0[0:64], Wp0[64:128], Wp0[128:192]
    W_g0s, W_g0r, W_ee = Wp0[192:256], Wp0[256:320], Wp0[320:384]
    (Un0, bu0) = p['node_proc_mlp'][0]
    U_ln, U_agg, U_en, U_ga = Un0[0:64], Un0[64:128], Un0[128:192], Un0[192:256]
    (w1n, b1n), (w2n, b2n), (w3n, b3n) = p['node_enc_mlp'][1:]
    gn, betan = p['node_enc_ln']
    en, a0, b0, P2, Q2, c_n = _full_call(
        _enc_node_body,
        [nodes, parts0[0], parts0[1], Wn0[:128], Wn0[128:], bn0.reshape(1, LAT),
         w1n, b1n.reshape(1, LAT), w2n, b2n.reshape(1, LAT), w3n,
         b3n.reshape(1, LAT), gn.reshape(1, LAT), betan.reshape(1, LAT),
         W_g0s, W_g0r, W_sn, W_rn, U_en, U_ga, bu0.reshape(1, LAT)],
        [jax.ShapeDtypeStruct((N, LAT), jnp.float32)] * 6)

    g2a, g2b = _sc_gather(P2, Q2, senders, receivers)
    c_e = _ce_pass(ee, pack(g2a), pack(g2b), W_ee, bp0)

    (w1, b1), (w2, b2), (w3, b3) = p['node_proc_mlp'][1:]
    gp, betap = p['node_proc_ln']

    def step(_, carry):
        ln, le, a, b = carry
        ga, gb = _sc_gather(a, b, senders, receivers)
        le2 = _edge_mlp(le, pack(ga), pack(gb), c_e, W_le,
                        p['edge_proc_mlp'][1:], p['edge_proc_ln'])
        parts = _sc_scatter(unpack(le2), receivers, zeros)
        ln2, a2, b2_ = _full_call(
            _step_node_body,
            [ln, parts[0], parts[1], c_n, U_ln, U_agg,
             w1, b1.reshape(1, LAT), w2, b2.reshape(1, LAT), w3,
             b3.reshape(1, LAT), gp.reshape(1, LAT), betap.reshape(1, LAT),
             W_sn, W_rn],
            [jax.ShapeDtypeStruct((N, LAT), jnp.float32)] * 3)
        return (ln2, le2, a2, b2_)

    ln, le, _, _ = lax.fori_loop(0, num_processing_steps, step, (en, ee, a0, b0))

    (Wd0, bd0), (wd1, bd1), (wd2, bd2), (wd3, bd3) = p['dec_mlp']
    D_OUT = wd3.shape[1]
    dec = pl.pallas_call(
        _dec_body,
        grid=(N // 2000,),
        in_specs=[pl.BlockSpec((2000, LAT), lambda i: (i, 0))] +
                 [pl.BlockSpec(a.shape, lambda i: (0,) * a.ndim) for a in
                  [Wd0, bd0.reshape(1, LAT), wd1, bd1.reshape(1, LAT),
                   wd2, bd2.reshape(1, LAT), wd3, bd3.reshape(1, D_OUT)]],
        out_specs=pl.BlockSpec((2000, D_OUT), lambda i: (i, 0)),
        out_shape=jax.ShapeDtypeStruct((N, D_OUT), jnp.float32),
    )(ln, Wd0, bd0.reshape(1, LAT), wd1, bd1.reshape(1, LAT), wd2,
      bd2.reshape(1, LAT), wd3, bd3.reshape(1, D_OUT))
    return dec
